# Initial kernel scaffold; baseline (speedup 1.0000x reference)
import jax
import jax.numpy as jnp
from jax.experimental import pallas as pl


def kernel(inputs, token_table, position_table):
    raise NotImplementedError("write your pallas kernel here")


# speedup vs baseline: 2.8542x; 2.8542x over previous
import functools

import jax
import jax.numpy as jnp
from jax import lax
from jax.experimental import pallas as pl
from jax.experimental.pallas import tpu as pltpu
from jax.experimental.pallas import tpu_sc as plsc

_CHUNK = 128
_NBUF = 4


@functools.cache
def _build(batch, seq, vocab, dim):
    info = plsc.get_sparse_core_info()
    nc, ns = info.num_cores, info.num_subcores
    nw = nc * ns
    rows = batch * seq
    assert rows % (nw * _CHUNK) == 0
    rows_per_w = rows // nw
    chunks_per_w = rows_per_w // _CHUNK
    assert rows_per_w % seq == 0
    assert chunks_per_w % _NBUF == 0
    assert dim % 16 == 0
    nvec = dim // 16
    pos_rows = seq + _CHUNK

    mesh = plsc.VectorSubcoreMesh(core_axis_name="c", subcore_axis_name="s")

    @functools.partial(
        pl.kernel,
        mesh=mesh,
        out_type=jax.ShapeDtypeStruct((rows, dim), jnp.float32),
        scratch_types=[
            pltpu.VMEM((chunks_per_w, _CHUNK), jnp.int32),
            pltpu.VMEM((pos_rows, dim), jnp.float32),
            pltpu.VMEM((_NBUF, _CHUNK, dim), jnp.float32),
        ] + [pltpu.SemaphoreType.DMA] * _NBUF,
        compiler_params=pltpu.CompilerParams(use_tc_tiling_on_sc=False),
    )
    def k(idx_hbm, tok_hbm, pos_hbm, out_hbm, idx_v, pos_v, rows_v, *sems):
        wid = lax.axis_index("s") * nc + lax.axis_index("c")
        pltpu.sync_copy(idx_hbm.at[pl.ds(wid * chunks_per_w, chunks_per_w)], idx_v)
        pltpu.sync_copy(pos_hbm, pos_v.at[pl.ds(0, seq)])
        pltpu.sync_copy(pos_hbm.at[pl.ds(0, _CHUNK)], pos_v.at[pl.ds(seq, _CHUNK)])

        def gather(g, b):
            return pltpu.make_async_copy(
                tok_hbm.at[idx_v.at[g]], rows_v.at[b], sems[b])

        for b in range(_NBUF):
            gather(b, b).start()

        def do_chunk(g, b):
            gather(g, b).wait()
            phi = lax.rem(g * _CHUNK, seq)

            def add_row(r, _):
                for j in range(nvec):
                    x = pos_v[phi + r, pl.ds(j * 16, 16)]
                    plsc.addupdate(rows_v.at[b, r, pl.ds(j * 16, 16)], x)
                return 0

            lax.fori_loop(0, _CHUNK, add_row, 0, unroll=4)
            row0 = wid * rows_per_w + g * _CHUNK
            pltpu.sync_copy(rows_v.at[b], out_hbm.at[pl.ds(row0, _CHUNK)])

            @pl.when(g + _NBUF < chunks_per_w)
            def _():
                gather(g + _NBUF, b).start()

        def outer(t, _):
            for b in range(_NBUF):
                do_chunk(t * _NBUF + b, b)
            return 0

        lax.fori_loop(0, chunks_per_w // _NBUF, outer, 0)

    return k


def kernel(inputs, token_table, position_table):
    batch, seq = inputs.shape
    vocab, dim = token_table.shape
    idx = inputs.astype(jnp.int32).reshape(-1).reshape(-1, _CHUNK)
    out = _build(batch, seq, vocab, dim)(idx, token_table, position_table)
    return out.reshape(batch, seq, dim)


# speedup vs baseline: 3.5450x; 1.2420x over previous
import functools

import jax
import jax.numpy as jnp
from jax import lax
from jax.experimental import pallas as pl
from jax.experimental.pallas import tpu as pltpu
from jax.experimental.pallas import tpu_sc as plsc

_NBUF = 4
_IMAX = 128


@functools.cache
def _build(batch, seq, vocab, dim):
    info = plsc.get_sparse_core_info()
    nc, ns = info.num_cores, info.num_subcores
    nw = nc * ns
    assert batch % nw == 0
    rows_per_w = batch // nw
    assert dim % 16 == 0
    nvec = dim // 16
    splits = [(0, min(seq, _IMAX))]
    if seq > _IMAX:
        splits.append((_IMAX, seq - _IMAX))
    assert all(o % 8 == 0 and n % 8 == 0 for o, n in splits)

    mesh = plsc.VectorSubcoreMesh(core_axis_name="c", subcore_axis_name="s")

    @functools.partial(
        pl.kernel,
        mesh=mesh,
        out_type=jax.ShapeDtypeStruct((batch, seq, dim), jnp.float32),
        scratch_types=[
            pltpu.VMEM((rows_per_w, seq), jnp.int32),
            pltpu.VMEM((seq, dim), jnp.float32),
            pltpu.VMEM((_NBUF, seq, dim), jnp.float32),
            pltpu.SemaphoreType.DMA((_NBUF,)),
            pltpu.SemaphoreType.DMA((_NBUF,)),
        ],
        compiler_params=pltpu.CompilerParams(use_tc_tiling_on_sc=False),
    )
    def k(idx_hbm, tok_hbm, pos_hbm, out_hbm, idx_v, pos_v, rows_v, gsem, wsem):
        wid = lax.axis_index("s") * nc + lax.axis_index("c")
        row0 = wid * rows_per_w
        pltpu.sync_copy(idx_hbm.at[pl.ds(row0, rows_per_w)], idx_v)
        pltpu.sync_copy(pos_hbm, pos_v)

        def gathers(r, slot):
            return [
                pltpu.make_async_copy(
                    tok_hbm.at[idx_v.at[r, pl.ds(o, n)]],
                    rows_v.at[slot, pl.ds(o, n)],
                    gsem.at[slot])
                for o, n in splits
            ]

        def write(r, slot):
            return pltpu.make_async_copy(
                rows_v.at[slot], out_hbm.at[row0 + r], wsem.at[slot])

        for r in range(2):
            for g in gathers(r, r):
                g.start()

        def body(r, _):
            slot = lax.rem(r, _NBUF)
            for g in gathers(r, slot):
                g.wait()
            for p0 in range(0, seq, 4):
                xs = [pos_v[p0 + q, pl.ds(j * 16, 16)]
                      for q in range(4) for j in range(nvec)]
                for i, x in enumerate(xs):
                    q, j = divmod(i, nvec)
                    plsc.addupdate(
                        rows_v.at[slot, p0 + q, pl.ds(j * 16, 16)], x)
            write(r, slot).start()
            slot2 = lax.rem(r + 2, _NBUF)

            @pl.when(r >= 2)
            def _():
                write(r - 2, slot2).wait()

            @pl.when(r + 2 < rows_per_w)
            def _():
                for g in gathers(r + 2, slot2):
                    g.start()

            return 0

        lax.fori_loop(0, rows_per_w, body, 0)
        for r in range(rows_per_w - 2, rows_per_w):
            write(r, r % _NBUF).wait()

    return k


def kernel(inputs, token_table, position_table):
    batch, seq = inputs.shape
    vocab, dim = token_table.shape
    idx = inputs.astype(jnp.int32)
    return _build(batch, seq, vocab, dim)(idx, token_table, position_table)
